# Initial kernel scaffold; baseline (speedup 1.0000x reference)
import jax
import jax.numpy as jnp
from jax.experimental import pallas as pl


def kernel(x, gate_w, W1, b1, mb, W2, b2, sW1, sb1, smb, sW2, sb2):
    raise NotImplementedError("write your pallas kernel here")


# speedup vs baseline: 1.8723x; 1.8723x over previous
import jax
import jax.numpy as jnp
from jax.experimental import pallas as pl
from jax.experimental.pallas import tpu as pltpu

E = 15
D = 768
FFN = 1024
TOPN = 3
SFFN = 2048
T = 2048
TT = 256
NT = T // TT
EP = 16


def _gelu_exact(z):
    return 0.5 * z * (1.0 + jax.lax.erf(z * 0.7071067811865476))


def _moe_body(gwt_ref, x_ref, W1_ref, b1_ref, mb_ref, W2_ref, b2_ref,
              y_ref, comb_ref):
    e = pl.program_id(0)
    t = pl.program_id(1)
    xt = x_ref[pl.ds(t * TT, TT), :]
    lane = jax.lax.broadcasted_iota(jnp.int32, (TT, EP), 1)

    @pl.when(e == 0)
    def _gate():
        logits = jnp.dot(xt, gwt_ref[...], preferred_element_type=jnp.float32)
        neg = jnp.float32(-1e30)
        logits = jnp.where(lane < E, logits, neg)
        m = jnp.max(logits, axis=1, keepdims=True)
        ex = jnp.exp(logits - m)
        p = ex / jnp.sum(ex, axis=1, keepdims=True)
        comb = jnp.zeros((TT, EP), jnp.float32)
        wsum = jnp.zeros((TT, 1), jnp.float32)
        picks = []
        pw = p
        for _ in range(TOPN):
            v = jnp.max(pw, axis=1, keepdims=True)
            i = jnp.min(jnp.where(pw >= v, lane, EP), axis=1, keepdims=True)
            hot = (lane == i).astype(jnp.float32)
            picks.append((v, hot))
            wsum = wsum + v
            pw = jnp.where(lane == i, neg, pw)
        inv = 1.0 / (wsum + 1e-20)
        for v, hot in picks:
            comb = comb + (v * inv) * hot
        comb_ref[pl.ds(t * TT, TT), :] = comb

    h = jnp.dot(xt, W1_ref[0], preferred_element_type=jnp.float32) + b1_ref[0]
    a = _gelu_exact(h[:, FFN:]) * h[:, :FFN] * mb_ref[0]
    eo = jnp.dot(a, W2_ref[0], preferred_element_type=jnp.float32) + b2_ref[0]
    c = jnp.sum(jnp.where(lane == e, comb_ref[pl.ds(t * TT, TT), :], 0.0),
                axis=1, keepdims=True)

    @pl.when(e == 0)
    def _init():
        y_ref[pl.ds(t * TT, TT), :] = c * eo

    @pl.when(e != 0)
    def _acc():
        y_ref[pl.ds(t * TT, TT), :] += c * eo


def _shared_body(x_ref, ymoe_ref, sW1_ref, sb1_ref, smb_ref, sW2_ref, sb2_ref,
                 y_ref):
    sh = jnp.dot(x_ref[...], sW1_ref[...], preferred_element_type=jnp.float32) + sb1_ref[...]
    sa = _gelu_exact(sh[:, SFFN:]) * sh[:, :SFFN] * smb_ref[...]
    so = jnp.dot(sa, sW2_ref[...], preferred_element_type=jnp.float32) + sb2_ref[...]
    y_ref[...] = ymoe_ref[...] + so


def kernel(x, gate_w, W1, b1, mb, W2, b2, sW1, sb1, smb, sW2, sb2):
    S, B, Dm = x.shape
    xf = x.reshape(T, D)
    gwt = jnp.pad(gate_w.T, ((0, 0), (0, EP - E)))

    y_moe = pl.pallas_call(
        _moe_body,
        grid=(E, NT),
        in_specs=[
            pl.BlockSpec((D, EP), lambda e, t: (0, 0)),
            pl.BlockSpec((T, D), lambda e, t: (0, 0)),
            pl.BlockSpec((1, D, 2 * FFN), lambda e, t: (e, 0, 0)),
            pl.BlockSpec((1, 1, 2 * FFN), lambda e, t: (e, 0, 0)),
            pl.BlockSpec((1, 1, FFN), lambda e, t: (e, 0, 0)),
            pl.BlockSpec((1, FFN, D), lambda e, t: (e, 0, 0)),
            pl.BlockSpec((1, 1, D), lambda e, t: (e, 0, 0)),
        ],
        out_specs=pl.BlockSpec((T, D), lambda e, t: (0, 0)),
        out_shape=jax.ShapeDtypeStruct((T, D), jnp.float32),
        scratch_shapes=[pltpu.VMEM((T, EP), jnp.float32)],
    )(gwt, xf, W1, b1.reshape(E, 1, 2 * FFN), mb.reshape(E, 1, FFN), W2,
      b2.reshape(E, 1, D))

    y = pl.pallas_call(
        _shared_body,
        grid=(NT,),
        in_specs=[
            pl.BlockSpec((TT, D), lambda t: (t, 0)),
            pl.BlockSpec((TT, D), lambda t: (t, 0)),
            pl.BlockSpec((D, 2 * SFFN), lambda t: (0, 0)),
            pl.BlockSpec((1, 2 * SFFN), lambda t: (0, 0)),
            pl.BlockSpec((1, SFFN), lambda t: (0, 0)),
            pl.BlockSpec((SFFN, D), lambda t: (0, 0)),
            pl.BlockSpec((1, D), lambda t: (0, 0)),
        ],
        out_specs=pl.BlockSpec((TT, D), lambda t: (t, 0)),
        out_shape=jax.ShapeDtypeStruct((T, D), jnp.float32),
    )(xf, y_moe, sW1, sb1.reshape(1, 2 * SFFN), smb.reshape(1, SFFN),
      sW2, sb2.reshape(1, D))

    return y.reshape(S, B, Dm)
